# baseline (device time: 213719 ns/iter reference)
import jax
import jax.numpy as jnp
from jax import lax
from jax.experimental import pallas as pl
from jax.experimental.pallas import tpu as pltpu

B, SQ, H, D = 4, 32, 8, 128
SKV = 4096
SCALE = D ** -0.5


KC = 1024
NKC = SKV // KC


def _local_attn_body(
    q_ref, k_ref, v_ref, o_ref, m_ref, l_ref, acc_o, acc_m, acc_l
):
    kc = pl.program_id(1)

    @pl.when(kc == 0)
    def _():
        acc_m[...] = jnp.full((H, SQ), -jnp.inf, jnp.float32)
        acc_l[...] = jnp.zeros((H, SQ), jnp.float32)
        acc_o[...] = jnp.zeros((SQ, H * D), jnp.float32)

    for h in range(H):
        sl = slice(h * D, (h + 1) * D)
        q = q_ref[0, :, sl]
        k = k_ref[0, :, sl]
        v = v_ref[0, :, sl]
        st = lax.dot_general(
            k, q.T, (((1,), (0,)), ((), ())), preferred_element_type=jnp.float32
        ) * SCALE
        m_prev = acc_m[h : h + 1, :]
        m_cur = jnp.max(st, axis=0, keepdims=True)
        m_new = jnp.maximum(m_prev, m_cur)
        alpha = jnp.exp(m_prev - m_new)
        pt = jnp.exp(st - m_new)
        l_new = acc_l[h : h + 1, :] * alpha + jnp.sum(pt, axis=0, keepdims=True)
        o_new = acc_o[:, sl] * alpha.T + lax.dot_general(
            pt.T, v, (((1,), (0,)), ((), ())), preferred_element_type=jnp.float32
        )
        acc_m[h : h + 1, :] = m_new
        acc_l[h : h + 1, :] = l_new
        acc_o[:, sl] = o_new

    @pl.when(kc == NKC - 1)
    def _():
        o_ref[0, :, :] = acc_o[...]
        m_ref[0, :, :] = acc_m[...].T
        l_ref[0, :, :] = acc_l[...].T


def _combine_body(
    o_ref, m_ref, l_ref, out_ref, ro_ref, rm_ref, rl_ref, send_sems, recv_sems
):
    my_x = lax.axis_index("x")
    my_y = lax.axis_index("y")
    my_z = lax.axis_index("z")
    nbr = (my_x, 1 - my_y, my_z)

    barrier = pltpu.get_barrier_semaphore()
    pl.semaphore_signal(
        barrier, inc=1, device_id=nbr, device_id_type=pl.DeviceIdType.MESH
    )
    pl.semaphore_wait(barrier, 1)

    copies = []
    for i, (src, dst) in enumerate(
        ((o_ref, ro_ref), (m_ref, rm_ref), (l_ref, rl_ref))
    ):
        c = pltpu.make_async_remote_copy(
            src_ref=src,
            dst_ref=dst,
            send_sem=send_sems.at[i],
            recv_sem=recv_sems.at[i],
            device_id=nbr,
            device_id_type=pl.DeviceIdType.MESH,
        )
        c.start()
        copies.append(c)
    for c in copies:
        c.wait()

    m1 = m_ref[...]
    l1 = l_ref[...]
    m2 = rm_ref[...]
    l2 = rl_ref[...]
    mx = jnp.maximum(m1, m2)
    a1 = jnp.exp(m1 - mx)
    a2 = jnp.exp(m2 - mx)
    denom = a1 * l1 + a2 * l2
    w1 = (a1 / denom)[..., None]
    w2 = (a2 / denom)[..., None]
    out_ref[...] = w1 * o_ref[...] + w2 * ro_ref[...]


def kernel(Q, K, V):
    q2 = Q.reshape(B, SQ, H * D)
    k2 = K.reshape(B, SKV, H * D)
    v2 = V.reshape(B, SKV, H * D)

    o_un, m, l = pl.pallas_call(
        _local_attn_body,
        grid=(B, NKC),
        in_specs=[
            pl.BlockSpec((1, SQ, H * D), lambda b, kc: (b, 0, 0)),
            pl.BlockSpec((1, KC, H * D), lambda b, kc: (b, kc, 0)),
            pl.BlockSpec((1, KC, H * D), lambda b, kc: (b, kc, 0)),
        ],
        out_specs=[
            pl.BlockSpec((1, SQ, H * D), lambda b, kc: (b, 0, 0)),
            pl.BlockSpec((1, SQ, H), lambda b, kc: (b, 0, 0)),
            pl.BlockSpec((1, SQ, H), lambda b, kc: (b, 0, 0)),
        ],
        out_shape=[
            jax.ShapeDtypeStruct((B, SQ, H * D), jnp.float32),
            jax.ShapeDtypeStruct((B, SQ, H), jnp.float32),
            jax.ShapeDtypeStruct((B, SQ, H), jnp.float32),
        ],
        scratch_shapes=[
            pltpu.VMEM((SQ, H * D), jnp.float32),
            pltpu.VMEM((H, SQ), jnp.float32),
            pltpu.VMEM((H, SQ), jnp.float32),
        ],
    )(q2, k2, v2)

    out = pl.pallas_call(
        _combine_body,
        out_shape=jax.ShapeDtypeStruct((B, SQ, H, D), jnp.float32),
        in_specs=[pl.BlockSpec(memory_space=pltpu.VMEM)] * 3,
        out_specs=pl.BlockSpec(memory_space=pltpu.VMEM),
        scratch_shapes=[
            pltpu.VMEM((B, SQ, H, D), jnp.float32),
            pltpu.VMEM((B, SQ, H), jnp.float32),
            pltpu.VMEM((B, SQ, H), jnp.float32),
            pltpu.SemaphoreType.DMA((3,)),
            pltpu.SemaphoreType.DMA((3,)),
        ],
        compiler_params=pltpu.CompilerParams(collective_id=0),
    )(o_un.reshape(B, SQ, H, D), m, l)
    return out


# device time: 85794 ns/iter; 2.4911x vs baseline; 2.4911x over previous
import jax
import jax.numpy as jnp
from jax import lax
from jax.experimental import pallas as pl
from jax.experimental.pallas import tpu as pltpu

B, SQ, H, D = 4, 32, 8, 128
SKV = 4096
SCALE = D ** -0.5

KC = 1024
NKC = SKV // KC


def _local_attn_body(
    q_ref, k_ref, v_ref, o_ref, m_ref, l_ref, acc_o, acc_m, acc_l
):
    kc = pl.program_id(1)

    @pl.when(kc == 0)
    def _():
        acc_m[...] = jnp.full((H, SQ), -jnp.inf, jnp.float32)
        acc_l[...] = jnp.zeros((H, SQ), jnp.float32)
        acc_o[...] = jnp.zeros((H, SQ, D), jnp.float32)

    q4 = q_ref[0]
    k4 = k_ref[0]
    v4 = v_ref[0]

    st = lax.dot_general(
        k4, q4, (((2,), (2,)), ((1,), (1,))),
        preferred_element_type=jnp.float32,
    ) * SCALE
    m_prev = acc_m[...]
    m_cur = jnp.max(st, axis=1)
    m_new = jnp.maximum(m_prev, m_cur)
    alpha = jnp.exp(m_prev - m_new)
    pt = jnp.exp(st - m_new[:, None, :])
    l_new = acc_l[...] * alpha + jnp.sum(pt, axis=1)
    pv = lax.dot_general(
        pt, v4, (((1,), (0,)), ((0,), (1,))),
        preferred_element_type=jnp.float32,
    )
    acc_o[...] = acc_o[...] * alpha[:, :, None] + pv
    acc_m[...] = m_new
    acc_l[...] = l_new

    @pl.when(kc == NKC - 1)
    def _():
        o_ref[0] = jnp.swapaxes(acc_o[...], 0, 1)
        m_ref[0] = acc_m[...]
        l_ref[0] = acc_l[...]


def _combine_body(
    o_ref, m_ref, l_ref, out_ref, ro_ref, rm_ref, rl_ref, send_sems, recv_sems
):
    my_x = lax.axis_index("x")
    my_y = lax.axis_index("y")
    my_z = lax.axis_index("z")
    nbr = (my_x, 1 - my_y, my_z)

    barrier = pltpu.get_barrier_semaphore()
    pl.semaphore_signal(
        barrier, inc=1, device_id=nbr, device_id_type=pl.DeviceIdType.MESH
    )
    pl.semaphore_wait(barrier, 1)

    copies = []
    for i, (src, dst) in enumerate(
        ((o_ref, ro_ref), (m_ref, rm_ref), (l_ref, rl_ref))
    ):
        c = pltpu.make_async_remote_copy(
            src_ref=src,
            dst_ref=dst,
            send_sem=send_sems.at[i],
            recv_sem=recv_sems.at[i],
            device_id=nbr,
            device_id_type=pl.DeviceIdType.MESH,
        )
        c.start()
        copies.append(c)
    for c in copies:
        c.wait()

    m1 = m_ref[...]
    l1 = l_ref[...]
    m2 = rm_ref[...]
    l2 = rl_ref[...]
    mx = jnp.maximum(m1, m2)
    a1 = jnp.exp(m1 - mx)
    a2 = jnp.exp(m2 - mx)
    denom = a1 * l1 + a2 * l2
    w1 = jnp.swapaxes(a1 / denom, 1, 2)[..., None]
    w2 = jnp.swapaxes(a2 / denom, 1, 2)[..., None]
    out_ref[...] = w1 * o_ref[...] + w2 * ro_ref[...]


def kernel(Q, K, V):
    o_un, m, l = pl.pallas_call(
        _local_attn_body,
        grid=(B, NKC),
        in_specs=[
            pl.BlockSpec((1, SQ, H, D), lambda b, kc: (b, 0, 0, 0)),
            pl.BlockSpec((1, KC, H, D), lambda b, kc: (b, kc, 0, 0)),
            pl.BlockSpec((1, KC, H, D), lambda b, kc: (b, kc, 0, 0)),
        ],
        out_specs=[
            pl.BlockSpec((1, SQ, H, D), lambda b, kc: (b, 0, 0, 0)),
            pl.BlockSpec((1, H, SQ), lambda b, kc: (b, 0, 0)),
            pl.BlockSpec((1, H, SQ), lambda b, kc: (b, 0, 0)),
        ],
        out_shape=[
            jax.ShapeDtypeStruct((B, SQ, H, D), jnp.float32),
            jax.ShapeDtypeStruct((B, H, SQ), jnp.float32),
            jax.ShapeDtypeStruct((B, H, SQ), jnp.float32),
        ],
        scratch_shapes=[
            pltpu.VMEM((H, SQ, D), jnp.float32),
            pltpu.VMEM((H, SQ), jnp.float32),
            pltpu.VMEM((H, SQ), jnp.float32),
        ],
    )(Q, K, V)

    return pl.pallas_call(
        _combine_body,
        out_shape=jax.ShapeDtypeStruct((B, SQ, H, D), jnp.float32),
        in_specs=[pl.BlockSpec(memory_space=pltpu.VMEM)] * 3,
        out_specs=pl.BlockSpec(memory_space=pltpu.VMEM),
        scratch_shapes=[
            pltpu.VMEM((B, SQ, H, D), jnp.float32),
            pltpu.VMEM((B, H, SQ), jnp.float32),
            pltpu.VMEM((B, H, SQ), jnp.float32),
            pltpu.SemaphoreType.DMA((3,)),
            pltpu.SemaphoreType.DMA((3,)),
        ],
        compiler_params=pltpu.CompilerParams(collective_id=0),
    )(o_un, m, l)


# device time: 69159 ns/iter; 3.0903x vs baseline; 1.2405x over previous
import jax
import jax.numpy as jnp
from jax import lax
from jax.experimental import pallas as pl
from jax.experimental.pallas import tpu as pltpu

B, SQ, H, D = 4, 32, 8, 128
SKV = 4096
SCALE = D ** -0.5

KC = 1024
NKC = SKV // KC
FLOOR = False


def _body(
    q_ref, k_ref, v_ref, out_ref,
    acc_o, acc_m, acc_l,
    obuf, stats, robuf, rstats,
    o_send_sems, o_recv_sems, s_send_sems, s_recv_sems,
):
    b = pl.program_id(0)
    kc = pl.program_id(1)
    my_x = lax.axis_index("x")
    my_y = lax.axis_index("y")
    my_z = lax.axis_index("z")
    nbr = (my_x, 1 - my_y, my_z)

    @pl.when(kc == 0)
    def _():
        acc_m[...] = jnp.full((H, SQ), -jnp.inf, jnp.float32)
        acc_l[...] = jnp.zeros((H, SQ), jnp.float32)
        acc_o[...] = jnp.zeros((H, SQ, D), jnp.float32)

    q4 = (q_ref[0] * SCALE).astype(jnp.bfloat16)
    k4 = k_ref[0].astype(jnp.bfloat16)
    v4 = v_ref[0].astype(jnp.bfloat16)

    if FLOOR:
        acc_o[...] = (
            acc_o[...]
            + jnp.swapaxes(k4.astype(jnp.float32)[:SQ], 0, 1)
            + jnp.swapaxes(v4.astype(jnp.float32)[:SQ], 0, 1)
        )
    else:
        st = lax.dot_general(
            k4, q4, (((2,), (2,)), ((1,), (1,))),
            preferred_element_type=jnp.float32,
        )
        m_prev = acc_m[...]
        m_cur = jnp.max(st, axis=1)
        m_new = jnp.maximum(m_prev, m_cur)
        alpha = jnp.exp(m_prev - m_new)
        pt = jnp.exp(st - m_new[:, None, :])
        l_new = acc_l[...] * alpha + jnp.sum(pt, axis=1)
        pv = lax.dot_general(
            pt.astype(jnp.bfloat16), v4, (((1,), (0,)), ((0,), (1,))),
            preferred_element_type=jnp.float32,
        )
        acc_o[...] = acc_o[...] * alpha[:, :, None] + pv
        acc_m[...] = m_new
        acc_l[...] = l_new

    def send(bb):
        co = pltpu.make_async_remote_copy(
            src_ref=obuf.at[bb],
            dst_ref=robuf.at[bb],
            send_sem=o_send_sems.at[bb],
            recv_sem=o_recv_sems.at[bb],
            device_id=nbr,
            device_id_type=pl.DeviceIdType.MESH,
        )
        cs = pltpu.make_async_remote_copy(
            src_ref=stats.at[bb],
            dst_ref=rstats.at[bb],
            send_sem=s_send_sems.at[bb],
            recv_sem=s_recv_sems.at[bb],
            device_id=nbr,
            device_id_type=pl.DeviceIdType.MESH,
        )
        return co, cs

    @pl.when(kc == NKC - 1)
    def _():
        obuf[b] = jnp.swapaxes(acc_o[...], 0, 1)
        stats[b, 0] = acc_m[...]
        stats[b, 1] = acc_l[...]

        @pl.when(b == 0)
        def _():
            barrier = pltpu.get_barrier_semaphore()
            pl.semaphore_signal(
                barrier, inc=1, device_id=nbr,
                device_id_type=pl.DeviceIdType.MESH,
            )
            pl.semaphore_wait(barrier, 1)

        for bb in range(B):
            @pl.when(b == bb)
            def _():
                co, cs = send(bb)
                co.start()
                cs.start()

    @pl.when((b == B - 1) & (kc == NKC - 1))
    def _():
        for bb in range(B):
            co, cs = send(bb)
            co.wait()
            cs.wait()

        m1 = stats[:, 0]
        l1 = stats[:, 1]
        m2 = rstats[:, 0]
        l2 = rstats[:, 1]
        mx = jnp.maximum(m1, m2)
        a1 = jnp.exp(m1 - mx)
        a2 = jnp.exp(m2 - mx)
        denom = a1 * l1 + a2 * l2
        w1 = jnp.swapaxes(a1 / denom, 1, 2)[..., None]
        w2 = jnp.swapaxes(a2 / denom, 1, 2)[..., None]
        out_ref[...] = w1 * obuf[...] + w2 * robuf[...]


def kernel(Q, K, V):
    return pl.pallas_call(
        _body,
        grid=(B, NKC),
        in_specs=[
            pl.BlockSpec((1, SQ, H, D), lambda b, kc: (b, 0, 0, 0)),
            pl.BlockSpec((1, KC, H, D), lambda b, kc: (b, kc, 0, 0)),
            pl.BlockSpec((1, KC, H, D), lambda b, kc: (b, kc, 0, 0)),
        ],
        out_specs=pl.BlockSpec(
            (B, SQ, H, D), lambda b, kc: (0, 0, 0, 0)
        ),
        out_shape=jax.ShapeDtypeStruct((B, SQ, H, D), jnp.float32),
        scratch_shapes=[
            pltpu.VMEM((H, SQ, D), jnp.float32),
            pltpu.VMEM((H, SQ), jnp.float32),
            pltpu.VMEM((H, SQ), jnp.float32),
            pltpu.VMEM((B, SQ, H, D), jnp.float32),
            pltpu.VMEM((B, 2, H, SQ), jnp.float32),
            pltpu.VMEM((B, SQ, H, D), jnp.float32),
            pltpu.VMEM((B, 2, H, SQ), jnp.float32),
            pltpu.SemaphoreType.DMA((B,)),
            pltpu.SemaphoreType.DMA((B,)),
            pltpu.SemaphoreType.DMA((B,)),
            pltpu.SemaphoreType.DMA((B,)),
        ],
        compiler_params=pltpu.CompilerParams(collective_id=0),
    )(Q, K, V)
